# baseline (speedup 1.0000x reference)
import functools

import jax
import jax.numpy as jnp
from jax.experimental import pallas as pl
from jax.experimental.pallas import tpu as pltpu

_MAX_VAL = 52000.0
_MIN_VAL = -53000.0
_NUM_LEVELS = 1000
_LEVELS_PAD = 1024
_N = 4
_C = 24
_T = 256
_D = 4096


def _roll_lanes(x, shift):
    return jnp.concatenate([x[:, -shift:], x[:, :-shift]], axis=1)


def _body(inT_ref, L_ref, F_ref, out_ref, smp_ref):
    c = pl.program_id(0)
    xcol = inT_ref[0]
    y = (xcol - _MIN_VAL) / (_MAX_VAL - _MIN_VAL) * (_NUM_LEVELS - 1)
    idx = jnp.clip(jnp.round(y), 0, _NUM_LEVELS - 1).astype(jnp.int32)
    lvl = jax.lax.broadcasted_iota(jnp.int32, (_T, _LEVELS_PAD), 1)
    oh = (idx == lvl).astype(jnp.int8)
    g = jnp.dot(oh, L_ref[...], preferred_element_type=jnp.int32)
    prod = g.astype(jnp.bfloat16) * F_ref[...]
    s = jnp.sum(prod, axis=0, keepdims=True).astype(jnp.float32)
    smp_ref[pl.ds(c, 1), :] = jnp.where(s > 0, 1.0, -1.0)

    @pl.when(c == _C - 1)
    def _():
        qa = smp_ref[...]
        r3 = _roll_lanes(qa, 3)
        r2 = _roll_lanes(qa, 2)
        r1 = _roll_lanes(qa, 1)
        w = (r3[0 : _C - 3] * r2[1 : _C - 2]) * (r1[2 : _C - 1] * qa[3:_C])
        s2 = jnp.sum(w, axis=0, keepdims=True)
        out_ref[...] = jnp.where(s2 > 0, 1.0, -1.0)


@jax.jit
def kernel(input, level_weight, features_weight):
    inT = input[:, :, None]
    Lp = jnp.pad(level_weight, ((0, _LEVELS_PAD - _NUM_LEVELS), (0, 0)))
    Lp = Lp.astype(jnp.int8)
    F = features_weight.astype(jnp.bfloat16)
    out = pl.pallas_call(
        _body,
        grid=(_C,),
        in_specs=[
            pl.BlockSpec((1, _T, 1), lambda c: (c, 0, 0)),
            pl.BlockSpec((_LEVELS_PAD, _D), lambda c: (0, 0)),
            pl.BlockSpec((_T, _D), lambda c: (0, 0)),
        ],
        out_specs=pl.BlockSpec((1, _D), lambda c: (0, 0)),
        out_shape=jax.ShapeDtypeStruct((1, _D), jnp.float32),
        scratch_shapes=[pltpu.VMEM((_C, _D), jnp.float32)],
    )(inT, Lp, F)
    return out


# speedup vs baseline: 1.1317x; 1.1317x over previous
import jax
import jax.numpy as jnp
from jax.experimental import pallas as pl
from jax.experimental.pallas import tpu as pltpu

_MAX_VAL = 52000.0
_MIN_VAL = -53000.0
_NUM_LEVELS = 1000
_LEVELS_PAD = 1024
_C = 24
_T = 256
_P = _T // 2
_D = 4096
_W = 128.0


def _roll_lanes(x, shift):
    return jnp.concatenate([x[:, -shift:], x[:, :-shift]], axis=1)


def _quant(x):
    y = (x - _MIN_VAL) / (_MAX_VAL - _MIN_VAL) * (_NUM_LEVELS - 1)
    return jnp.clip(jnp.round(y), 0, _NUM_LEVELS - 1).astype(jnp.int32)


def _body(in_ref, L_ref, Gm_ref, Fo_ref, out_ref, smp_ref):
    c = pl.program_id(0)
    idx_e = _quant(in_ref[0, :, 0:1])
    idx_o = _quant(in_ref[0, :, 1:2])
    lvl = jax.lax.broadcasted_iota(jnp.int32, (_P, _LEVELS_PAD), 1)
    oh = (idx_e == lvl).astype(jnp.bfloat16) + (idx_o == lvl).astype(
        jnp.bfloat16
    ) * jnp.bfloat16(1.0 / _W)
    g = jnp.dot(oh, L_ref[...], preferred_element_type=jnp.float32)
    mask = g > 0
    s = jnp.sum(jnp.where(mask, Gm_ref[...], -Gm_ref[...]) + g * Fo_ref[...],
                axis=0, keepdims=True)
    smp_ref[pl.ds(c, 1), :] = jnp.where(s > 0, 1.0, -1.0)

    @pl.when(c == _C - 1)
    def _():
        qa = smp_ref[...]
        r3 = _roll_lanes(qa, 3)
        r2 = _roll_lanes(qa, 2)
        r1 = _roll_lanes(qa, 1)
        w = (r3[0 : _C - 3] * r2[1 : _C - 2]) * (r1[2 : _C - 1] * qa[3:_C])
        s2 = jnp.sum(w, axis=0, keepdims=True)
        out_ref[...] = jnp.where(s2 > 0, 1.0, -1.0)


@jax.jit
def kernel(input, level_weight, features_weight):
    x3 = jnp.reshape(input, (_C, _P, 2))
    Lp = jnp.pad(level_weight, ((0, _LEVELS_PAD - _NUM_LEVELS), (0, 0)))
    Lp = Lp.astype(jnp.bfloat16)
    F3 = jnp.reshape(features_weight, (_P, 2, _D))
    Fo = F3[:, 1, :] * _W
    Gm = F3[:, 0, :] - Fo
    out = pl.pallas_call(
        _body,
        grid=(_C,),
        in_specs=[
            pl.BlockSpec((1, _P, 2), lambda c: (c, 0, 0)),
            pl.BlockSpec((_LEVELS_PAD, _D), lambda c: (0, 0)),
            pl.BlockSpec((_P, _D), lambda c: (0, 0)),
            pl.BlockSpec((_P, _D), lambda c: (0, 0)),
        ],
        out_specs=pl.BlockSpec((1, _D), lambda c: (0, 0)),
        out_shape=jax.ShapeDtypeStruct((1, _D), jnp.float32),
        scratch_shapes=[pltpu.VMEM((_C, _D), jnp.float32)],
    )(x3, Lp, Gm, Fo)
    return out


# speedup vs baseline: 1.6210x; 1.4324x over previous
import jax
import jax.numpy as jnp
from jax.experimental import pallas as pl
from jax.experimental.pallas import tpu as pltpu

_MAX_VAL = 52000.0
_MIN_VAL = -53000.0
_NUM_LEVELS = 1000
_LEVELS_PAD = 1024
_C = 24
_T = 256
_P = _T // 2
_D = 4096
_W = 128.0


def _roll_lanes(x, shift):
    return jnp.concatenate([x[:, -shift:], x[:, :-shift]], axis=1)


def _quant(x):
    y = (x - _MIN_VAL) / (_MAX_VAL - _MIN_VAL) * (_NUM_LEVELS - 1)
    return jnp.clip(jnp.round(y), 0, _NUM_LEVELS - 1).astype(jnp.int32)


def _body(in_ref, L_ref, F_ref, out_ref, Lbf_ref, Gm_ref, Fo_ref, smp_ref):
    c = pl.program_id(0)

    @pl.when(c == 0)
    def _():
        Lbf_ref[0:_NUM_LEVELS, :] = L_ref[...].astype(jnp.bfloat16)
        Lbf_ref[_NUM_LEVELS:_LEVELS_PAD, :] = jnp.zeros(
            (_LEVELS_PAD - _NUM_LEVELS, _D), jnp.bfloat16
        )
        fo = F_ref[:, 1, :] * _W
        Fo_ref[...] = fo
        Gm_ref[...] = F_ref[:, 0, :] - fo

    idx_e = _quant(in_ref[0, :, 0:1])
    idx_o = _quant(in_ref[0, :, 1:2])
    lvl = jax.lax.broadcasted_iota(jnp.int32, (_P, _LEVELS_PAD), 1)
    oh = (idx_e == lvl).astype(jnp.bfloat16) + (idx_o == lvl).astype(
        jnp.bfloat16
    ) * jnp.bfloat16(1.0 / _W)
    g = jnp.dot(oh, Lbf_ref[...], preferred_element_type=jnp.float32)
    mask = g > 0
    s = jnp.sum(jnp.where(mask, Gm_ref[...], -Gm_ref[...]) + g * Fo_ref[...],
                axis=0, keepdims=True)
    smp_ref[pl.ds(c, 1), :] = jnp.where(s > 0, 1.0, -1.0)

    @pl.when(c == _C - 1)
    def _():
        qa = smp_ref[...]
        r3 = _roll_lanes(qa, 3)
        r2 = _roll_lanes(qa, 2)
        r1 = _roll_lanes(qa, 1)
        w = (r3[0 : _C - 3] * r2[1 : _C - 2]) * (r1[2 : _C - 1] * qa[3:_C])
        s2 = jnp.sum(w, axis=0, keepdims=True)
        out_ref[...] = jnp.where(s2 > 0, 1.0, -1.0)


@jax.jit
def kernel(input, level_weight, features_weight):
    x3 = jnp.reshape(input, (_C, _P, 2))
    F3 = jnp.reshape(features_weight, (_P, 2, _D))
    out = pl.pallas_call(
        _body,
        grid=(_C,),
        in_specs=[
            pl.BlockSpec((1, _P, 2), lambda c: (c, 0, 0)),
            pl.BlockSpec((_NUM_LEVELS, _D), lambda c: (0, 0)),
            pl.BlockSpec((_P, 2, _D), lambda c: (0, 0, 0)),
        ],
        out_specs=pl.BlockSpec((1, _D), lambda c: (0, 0)),
        out_shape=jax.ShapeDtypeStruct((1, _D), jnp.float32),
        scratch_shapes=[
            pltpu.VMEM((_LEVELS_PAD, _D), jnp.bfloat16),
            pltpu.VMEM((_P, _D), jnp.float32),
            pltpu.VMEM((_P, _D), jnp.float32),
            pltpu.VMEM((_C, _D), jnp.float32),
        ],
    )(x3, level_weight, F3)
    return out
